# Initial kernel scaffold; baseline (speedup 1.0000x reference)
import jax
import jax.numpy as jnp
from jax.experimental import pallas as pl


def kernel(x, theta, H, edge_index, W1, b1, W2, b2, W3, b3):
    raise NotImplementedError("write your pallas kernel here")


# speedup vs baseline: 10.8314x; 10.8314x over previous
import functools

import jax
import jax.numpy as jnp
from jax import lax
from jax.experimental import pallas as pl
from jax.experimental.pallas import tpu as pltpu
from jax.experimental.pallas import tpu_sc as plsc

N = 100000
E = 1600000
S = 8
HID = 32

NC = 2
NS = 16
NW = NC * NS
PER_W = E // NW
CG = 1000
CS = 1000
NPS = N // NS

def _mesh():
    return plsc.VectorSubcoreMesh(core_axis_name="c", subcore_axis_name="s",
                                  num_cores=NC, num_subcores=NS)


@functools.cache
def _make_sc_gather():
    @functools.partial(
        pl.kernel,
        mesh=_mesh(),
        out_type=[
            jax.ShapeDtypeStruct((E, 4), jnp.float32),
            jax.ShapeDtypeStruct((E, 20), jnp.float32),
        ],
        scratch_types=[
            pltpu.VMEM((CG,), jnp.int32),
            pltpu.VMEM((CG,), jnp.int32),
            pltpu.VMEM((CG, 4), jnp.float32),
            pltpu.VMEM((CG, 20), jnp.float32),
            pltpu.SemaphoreType.DMA,
            pltpu.SemaphoreType.DMA,
        ],
        compiler_params=pltpu.CompilerParams(use_tc_tiling_on_sc=False),
    )
    def _sc_gather(src_hbm, dst_hbm, ii_hbm, jj_hbm, out_i, out_j,
                   ii_v, jj_v, ri_v, rj_v, s1, s2):
        wid = lax.axis_index("s") * NC + lax.axis_index("c")
        base = wid * PER_W

        def body(t, carry):
            off = base + t * CG
            pltpu.sync_copy(ii_hbm.at[pl.ds(off, CG)], ii_v)
            pltpu.sync_copy(jj_hbm.at[pl.ds(off, CG)], jj_v)
            ca = pltpu.async_copy(src_hbm.at[ii_v], ri_v, s1)
            cb = pltpu.async_copy(dst_hbm.at[jj_v], rj_v, s2)
            ca.wait()
            cb.wait()
            pltpu.sync_copy(ri_v, out_i.at[pl.ds(off, CG)])
            pltpu.sync_copy(rj_v, out_j.at[pl.ds(off, CG)])
            return carry

        lax.fori_loop(0, PER_W // CG, body, 0)

    return _sc_gather


@functools.cache
def _make_sc_scatter():
    @functools.partial(
        pl.kernel,
        mesh=_mesh(),
        out_type=jax.ShapeDtypeStruct((2 * N, 16), jnp.float32),
        scratch_types=[
            pltpu.VMEM((CS,), jnp.int32),
            pltpu.VMEM((CS, 16), jnp.float32),
            pltpu.VMEM_SHARED((N, 16), jnp.float32),
            pltpu.SemaphoreType.DMA,
        ],
        compiler_params=pltpu.CompilerParams(use_tc_tiling_on_sc=False),
    )
    def _sc_scatter(m_hbm, ii_hbm, z_hbm, out_hbm, ii_v, m_v, acc, sem):
        c = lax.axis_index("c")
        s = lax.axis_index("s")
        wid = s * NC + c
        pltpu.sync_copy(z_hbm, acc.at[pl.ds(s * NPS, NPS)])
        plsc.subcore_barrier()

        base = wid * PER_W

        def body(t, carry):
            off = base + t * CS
            pltpu.sync_copy(ii_hbm.at[pl.ds(off, CS)], ii_v)
            pltpu.sync_copy(m_hbm.at[pl.ds(off, CS)], m_v)
            pltpu.sync_copy(m_v, acc.at[ii_v], add=True)
            return carry

        lax.fori_loop(0, PER_W // CS, body, 0)
        plsc.subcore_barrier()
        pltpu.sync_copy(acc.at[pl.ds(s * NPS, NPS)],
                        out_hbm.at[pl.ds(c * N + s * NPS, NPS)])

    return _sc_scatter


BE = 512


def _silu(v):
    return v * (1.0 / (1.0 + jnp.exp(-v)))


def _mlp_body(ei_ref, ej_ref, w1_ref, b1_ref, w2_ref, b2_ref, w3_ref, b3_ref,
              out_ref):
    ei = ei_ref[...]
    ej = ej_ref[...]
    xi = ei[:, 0:3]
    ti = ei[:, 3:4]
    xj = ej[:, 0:3]
    tj = ej[:, 3:4]
    dx = xj - xi
    dxx = dx[:, 0:1]
    dxy = dx[:, 1:2]
    dz = dx[:, 2:3]
    rxy2 = dxx * dxx + dxy * dxy
    r3 = jnp.sqrt(rxy2 + dz * dz)
    rxy = jnp.sqrt(rxy2)
    cti = jnp.cos(ti)
    sti = jnp.sin(ti)
    d2 = 2.0 * (tj - ti)
    ca = jnp.cos(d2)
    sa = jnp.sin(d2)
    pos = rxy > 0.0
    inv = jnp.where(pos, 1.0, 0.0) / jnp.where(pos, rxy, 1.0)
    cdp = jnp.where(pos, (dxx * cti + dxy * sti) * inv, cti)
    sdp = jnp.where(pos, (dxy * cti - dxx * sti) * inv, -sti)
    sgn = jnp.sign(dz)
    geom = jnp.concatenate([r3, rxy, dz, sdp, cdp, sa, ca, sgn * sa], axis=1)
    w1 = w1_ref[...]
    gg = jnp.dot(geom, w1[2:10, :], preferred_element_type=jnp.float32)
    gg = gg + b1_ref[...]
    vx = ej[:, 4:12]
    vy = ej[:, 12:20]
    vxr = ca * vx - sa * vy
    vyr = sa * vx + ca * vy
    w1x = w1[0:1, :].reshape(1, 1, HID)
    w1y = w1[1:2, :].reshape(1, 1, HID)
    h1 = vxr[:, :, None] * w1x + vyr[:, :, None] * w1y + gg[:, None, :]
    h1 = _silu(h1.reshape(BE * S, HID))
    h2 = _silu(jnp.dot(h1, w2_ref[...], preferred_element_type=jnp.float32)
               + b2_ref[...])
    out_ref[...] = (jnp.dot(h2, w3_ref[...], preferred_element_type=jnp.float32)
                    + b3_ref[...])


def _run_mlp(ei, ej, W1, b1, W2, b2, W3, b3):
    grid = E // BE
    full = lambda t: (0, 0)
    return pl.pallas_call(
        _mlp_body,
        grid=(grid,),
        in_specs=[
            pl.BlockSpec((BE, 4), lambda t: (t, 0)),
            pl.BlockSpec((BE, 20), lambda t: (t, 0)),
            pl.BlockSpec((10, HID), full),
            pl.BlockSpec((1, HID), full),
            pl.BlockSpec((HID, HID), full),
            pl.BlockSpec((1, HID), full),
            pl.BlockSpec((HID, 2), full),
            pl.BlockSpec((1, 2), full),
        ],
        out_specs=pl.BlockSpec((BE * S, 2), lambda t: (t, 0)),
        out_shape=jax.ShapeDtypeStruct((E * S, 2), jnp.float32),
    )(ei, ej, W1, b1.reshape(1, HID), W2, b2.reshape(1, HID), W3,
      b3.reshape(1, 2))


BN = 1000


def _add_body(a_ref, b_ref, o_ref):
    o_ref[...] = a_ref[...] + b_ref[...]


def _combine(parts):
    grid = N // BN
    return pl.pallas_call(
        _add_body,
        grid=(grid,),
        in_specs=[
            pl.BlockSpec((BN, 16), lambda t: (t, 0)),
            pl.BlockSpec((BN, 16), lambda t: (t + N // BN, 0)),
        ],
        out_specs=pl.BlockSpec((BN, 16), lambda t: (t, 0)),
        out_shape=jax.ShapeDtypeStruct((N, 16), jnp.float32),
    )(parts, parts)


@jax.jit
def kernel(x, theta, H, edge_index, W1, b1, W2, b2, W3, b3):
    ii = edge_index[0].astype(jnp.int32)
    jj = edge_index[1].astype(jnp.int32)
    th = theta[:, None]
    src_tab = jnp.concatenate([x, th], axis=1)
    dst_tab = jnp.concatenate([x, th, H[:, :, 0], H[:, :, 1]], axis=1)
    ei, ej = _make_sc_gather()(src_tab, dst_tab, ii, jj)
    msg = _run_mlp(ei, ej, W1, b1, W2, b2, W3, b3)
    m16 = msg.reshape(E, 16)
    zeros = jnp.zeros((NPS, 16), jnp.float32)
    parts = _make_sc_scatter()(m16, ii, zeros)
    return _combine(parts).reshape(N, S, 2)
